# baseline (speedup 1.0000x reference)
import functools

import jax
import jax.numpy as jnp
from jax import lax
from jax.experimental import pallas as pl
from jax.experimental.pallas import tpu as pltpu
from jax.experimental.pallas import tpu_sc as plsc

D = 32
LW = 128
RPL = LW // D
L = 16
NC, NS = 2, 16
NW = NC * NS
K = 16
H = 2
TBLK = 512
OBLK = TBLK // RPL


def _lines_body(x_ref, s_ref, o_ref):
    x = x_ref[...].astype(jnp.bfloat16)
    s = s_ref[...]
    for j in range(RPL):
        sj = s[:, j * OBLK:(j + 1) * OBLK]
        tmp = lax.dot_general(x, sj, (((1,), (0,)), ((), ())),
                              preferred_element_type=jnp.float32)
        o_ref[:, j * D:(j + 1) * D] = tmp.T


def _to_lines(lat_t, s4):
    n = lat_t.shape[1]
    nlines = n // RPL
    grid = pl.cdiv(n, TBLK)
    return pl.pallas_call(
        _lines_body,
        grid=(grid,),
        in_specs=[pl.BlockSpec((D, TBLK), lambda i: (0, i)),
                  pl.BlockSpec((TBLK, TBLK), lambda i: (0, 0))],
        out_specs=pl.BlockSpec((OBLK, LW), lambda i: (i, 0)),
        out_shape=jax.ShapeDtypeStruct((nlines, LW), jnp.float32),
    )(lat_t, s4)


@functools.partial(jax.jit, static_argnames=("b",))
def _mf_sc(user, item, ulat_t, ilat_t, b):
    b_per_w = b // NW
    rows_h = b_per_w // H
    nblk = rows_h // K

    c = jnp.arange(TBLK)[:, None]
    m = jnp.arange(TBLK)[None, :]
    s4 = (c == RPL * (m % OBLK) + m // OBLK).astype(jnp.bfloat16)
    ulat4 = _to_lines(ulat_t, s4)
    ilat4 = _to_lines(ilat_t, s4)

    mesh = plsc.VectorSubcoreMesh(core_axis_name="c", subcore_axis_name="s")

    @functools.partial(
        pl.kernel,
        out_type=(
            jax.ShapeDtypeStruct((b * D,), jnp.float32),
            jax.ShapeDtypeStruct((b * D,), jnp.float32),
            jax.ShapeDtypeStruct((b,), jnp.float32),
        ),
        mesh=mesh,
        compiler_params=pltpu.CompilerParams(needs_layout_passes=False),
        scratch_types=[
            pltpu.VMEM((b // NW,), jnp.int32),
            pltpu.VMEM((b // NW,), jnp.int32),
            pltpu.VMEM((b // NW // H * LW,), jnp.float32),
            pltpu.VMEM((b // NW // H * LW,), jnp.float32),
            pltpu.VMEM((b // NW // H * D,), jnp.float32),
            pltpu.VMEM((b // NW // H * D,), jnp.float32),
            pltpu.VMEM((b // NW,), jnp.float32),
            pltpu.SemaphoreType.DMA,
            pltpu.SemaphoreType.DMA,
        ],
    )
    def k(user_h, item_h, ulat_h, ilat_h, out_u, out_i, out_r,
          idx_us, idx_is, u_lines, i_lines, u_rows, i_rows, res_v,
          gsem, wsem):
        wid = lax.axis_index("s") * NC + lax.axis_index("c")
        base = wid * b_per_w

        pltpu.sync_copy(user_h.at[pl.ds(base, b_per_w)], idx_us)
        pltpu.sync_copy(item_h.at[pl.ds(base, b_per_w)], idx_is)

        lane = lax.iota(jnp.int32, L)

        for h in range(H):
            hbase = h * rows_h

            def fire(blk, hbase=hbase):
                rr0 = blk * K
                uvec = idx_us[pl.ds(hbase + rr0, K)] // RPL
                ivec = idx_is[pl.ds(hbase + rr0, K)] // RPL
                for kk in range(K):
                    rr = rr0 + kk
                    pltpu.async_copy(
                        ulat_h.at[uvec[kk]],
                        u_lines.at[pl.ds(rr * LW, LW)], gsem)
                    pltpu.async_copy(
                        ilat_h.at[ivec[kk]],
                        i_lines.at[pl.ds(rr * LW, LW)], gsem)

            def drain_batch():
                pltpu.make_async_copy(
                    out_u.at[pl.ds(0, 2 * K * LW)],
                    u_lines.at[pl.ds(0, 2 * K * LW)], gsem).wait()

            fire(0)

            @pl.loop(1, nblk)
            def _(blk):
                fire(blk)
                drain_batch()

            drain_batch()

            @pl.loop(0, rows_h // L)
            def _(g, hbase=hbase):
                r0 = g * L
                uoff = (idx_us[pl.ds(hbase + r0, L)] % RPL) * D
                ioff = (idx_is[pl.ds(hbase + r0, L)] % RPL) * D
                for kk in range(L):
                    rr = r0 + kk
                    uo = rr * LW + uoff[kk]
                    io = rr * LW + ioff[kk]
                    u_rows[pl.ds(rr * D, L)] = u_lines[pl.ds(uo, L)]
                    u_rows[pl.ds(rr * D + L, L)] = u_lines[pl.ds(uo + L, L)]
                    i_rows[pl.ds(rr * D, L)] = i_lines[pl.ds(io, L)]
                    i_rows[pl.ds(rr * D + L, L)] = i_lines[pl.ds(io + L, L)]

            wb_u = pltpu.async_copy(
                u_rows, out_u.at[pl.ds((base + hbase) * D, rows_h * D)], wsem)
            wb_i = pltpu.async_copy(
                i_rows, out_i.at[pl.ds((base + hbase) * D, rows_h * D)], wsem)

            @pl.loop(0, rows_h // L)
            def _(g):
                r0 = g * L
                acc = jnp.zeros((L,), jnp.float32)
                for kk in range(L):
                    f0 = (r0 + kk) * D
                    t = (u_rows[pl.ds(f0, L)] * i_rows[pl.ds(f0, L)]
                         + u_rows[pl.ds(f0 + L, L)] * i_rows[pl.ds(f0 + L, L)])
                    acc = jnp.where(lane == kk, jnp.sum(t), acc)
                res_v[pl.ds(hbase + r0, L)] = acc

            wb_u.wait()
            wb_i.wait()

        pltpu.sync_copy(res_v, out_r.at[pl.ds(base, b_per_w)])

    return k(user, item, ulat4, ilat4)


def kernel(user, item, his_r, rct_r, user_bias_w, item_bias_w,
           user_laten_w, item_laten_w):
    b = user.shape[0]
    out_uf, out_if, res = _mf_sc(user, item, user_laten_w.T,
                                 item_laten_w.T, b)
    return out_uf.reshape(b, D), out_if.reshape(b, D), res


# speedup vs baseline: 1.0187x; 1.0187x over previous
import functools

import jax
import jax.numpy as jnp
from jax import lax
from jax.experimental import pallas as pl
from jax.experimental.pallas import tpu as pltpu
from jax.experimental.pallas import tpu_sc as plsc

D = 32
LW = 128
RPL = LW // D
L = 16
NC, NS = 2, 16
NW = NC * NS
K = 16
H = 2
TBLK = 512
OBLK = TBLK // RPL


def _lines_body(x_ref, s_ref, o_ref):
    x = x_ref[...].astype(jnp.bfloat16)
    s = s_ref[...]
    outs = []
    for j in range(RPL):
        sjt = s[j * OBLK:(j + 1) * OBLK, :]
        outs.append(lax.dot_general(sjt, x, (((1,), (1,)), ((), ())),
                                    preferred_element_type=jnp.float32))
    o_ref[...] = jnp.concatenate(outs, axis=1)


def _to_lines(lat_t, s4):
    n = lat_t.shape[1]
    nlines = n // RPL
    grid = pl.cdiv(n, TBLK)
    return pl.pallas_call(
        _lines_body,
        grid=(grid,),
        in_specs=[pl.BlockSpec((D, TBLK), lambda i: (0, i)),
                  pl.BlockSpec((TBLK, TBLK), lambda i: (0, 0))],
        out_specs=pl.BlockSpec((OBLK, LW), lambda i: (i, 0)),
        out_shape=jax.ShapeDtypeStruct((nlines, LW), jnp.float32),
    )(lat_t, s4)


@functools.partial(jax.jit, static_argnames=("b",))
def _mf_sc(user, item, ulat_t, ilat_t, b):
    b_per_w = b // NW
    rows_h = b_per_w // H
    nblk = rows_h // K

    c = jnp.arange(TBLK)[:, None]
    m = jnp.arange(TBLK)[None, :]
    s4 = (m == RPL * (c % OBLK) + c // OBLK).astype(jnp.bfloat16)
    ulat4 = _to_lines(ulat_t, s4)
    ilat4 = _to_lines(ilat_t, s4)

    mesh = plsc.VectorSubcoreMesh(core_axis_name="c", subcore_axis_name="s")

    @functools.partial(
        pl.kernel,
        out_type=(
            jax.ShapeDtypeStruct((b * D,), jnp.float32),
            jax.ShapeDtypeStruct((b * D,), jnp.float32),
            jax.ShapeDtypeStruct((b,), jnp.float32),
        ),
        mesh=mesh,
        compiler_params=pltpu.CompilerParams(needs_layout_passes=False),
        scratch_types=[
            pltpu.VMEM((b // NW,), jnp.int32),
            pltpu.VMEM((b // NW,), jnp.int32),
            pltpu.VMEM((b // NW // H * LW,), jnp.float32),
            pltpu.VMEM((b // NW // H * LW,), jnp.float32),
            pltpu.VMEM((b // NW // H * D,), jnp.float32),
            pltpu.VMEM((b // NW // H * D,), jnp.float32),
            pltpu.VMEM((b // NW,), jnp.float32),
            pltpu.SemaphoreType.DMA,
            pltpu.SemaphoreType.DMA,
        ],
    )
    def k(user_h, item_h, ulat_h, ilat_h, out_u, out_i, out_r,
          idx_us, idx_is, u_lines, i_lines, u_rows, i_rows, res_v,
          gsem, wsem):
        wid = lax.axis_index("s") * NC + lax.axis_index("c")
        base = wid * b_per_w

        pltpu.sync_copy(user_h.at[pl.ds(base, b_per_w)], idx_us)
        pltpu.sync_copy(item_h.at[pl.ds(base, b_per_w)], idx_is)

        lane = lax.iota(jnp.int32, L)

        for h in range(H):
            hbase = h * rows_h

            def fire(blk, hbase=hbase):
                rr0 = blk * K
                uvec = idx_us[pl.ds(hbase + rr0, K)] // RPL
                ivec = idx_is[pl.ds(hbase + rr0, K)] // RPL
                for kk in range(K):
                    rr = rr0 + kk
                    pltpu.async_copy(
                        ulat_h.at[uvec[kk]],
                        u_lines.at[pl.ds(rr * LW, LW)], gsem)
                    pltpu.async_copy(
                        ilat_h.at[ivec[kk]],
                        i_lines.at[pl.ds(rr * LW, LW)], gsem)

            def drain_batch():
                pltpu.make_async_copy(
                    out_u.at[pl.ds(0, 2 * K * LW)],
                    u_lines.at[pl.ds(0, 2 * K * LW)], gsem).wait()

            fire(0)

            @pl.loop(1, nblk)
            def _(blk):
                fire(blk)
                drain_batch()

            drain_batch()

            @pl.loop(0, rows_h // L)
            def _(g, hbase=hbase):
                r0 = g * L
                uoff = (idx_us[pl.ds(hbase + r0, L)] % RPL) * D
                ioff = (idx_is[pl.ds(hbase + r0, L)] % RPL) * D
                for kk in range(L):
                    rr = r0 + kk
                    uo = rr * LW + uoff[kk]
                    io = rr * LW + ioff[kk]
                    u_rows[pl.ds(rr * D, L)] = u_lines[pl.ds(uo, L)]
                    u_rows[pl.ds(rr * D + L, L)] = u_lines[pl.ds(uo + L, L)]
                    i_rows[pl.ds(rr * D, L)] = i_lines[pl.ds(io, L)]
                    i_rows[pl.ds(rr * D + L, L)] = i_lines[pl.ds(io + L, L)]

            wb_u = pltpu.async_copy(
                u_rows, out_u.at[pl.ds((base + hbase) * D, rows_h * D)], wsem)
            wb_i = pltpu.async_copy(
                i_rows, out_i.at[pl.ds((base + hbase) * D, rows_h * D)], wsem)

            @pl.loop(0, rows_h // L)
            def _(g):
                r0 = g * L
                acc = jnp.zeros((L,), jnp.float32)
                for kk in range(L):
                    f0 = (r0 + kk) * D
                    t = (u_rows[pl.ds(f0, L)] * i_rows[pl.ds(f0, L)]
                         + u_rows[pl.ds(f0 + L, L)] * i_rows[pl.ds(f0 + L, L)])
                    acc = jnp.where(lane == kk, jnp.sum(t), acc)
                res_v[pl.ds(hbase + r0, L)] = acc

            wb_u.wait()
            wb_i.wait()

        pltpu.sync_copy(res_v, out_r.at[pl.ds(base, b_per_w)])

    return k(user, item, ulat4, ilat4)


def kernel(user, item, his_r, rct_r, user_bias_w, item_bias_w,
           user_laten_w, item_laten_w):
    b = user.shape[0]
    out_uf, out_if, res = _mf_sc(user, item, user_laten_w.T,
                                 item_laten_w.T, b)
    return out_uf.reshape(b, D), out_if.reshape(b, D), res
